# Initial kernel scaffold; baseline (speedup 1.0000x reference)
import jax
import jax.numpy as jnp
from jax.experimental import pallas as pl


def kernel(x, edge_index, edge_attr, batch, params):
    raise NotImplementedError("write your pallas kernel here")


# speedup vs baseline: 1.7468x; 1.7468x over previous
import functools

import jax
import jax.numpy as jnp
from jax import lax
from jax.experimental import pallas as pl
from jax.experimental.pallas import tpu as pltpu
from jax.experimental.pallas import tpu_sc as plsc

N = 10000
E = 320000
C = 128
DE = 16
G = 64
L = 3

NB_N = 10
BN_ROWS = N // NB_N
BE_ROWS = 1000
NB_E = E // BE_ROWS

NW = 32
EPW = E // NW
K = 80
NCH = EPW // K
RPT = 624
RTAIL = N - 16 * RPT


def _embed_body(x_ref, w0_ref, w1_ref, o_ref):
    h = jnp.maximum(x_ref[...] @ w0_ref[...], 0.0)
    o_ref[...] = h @ w1_ref[...]


def _embed(x, w0, w1):
    return pl.pallas_call(
        _embed_body,
        grid=(NB_N,),
        in_specs=[pl.BlockSpec((BN_ROWS, C), lambda i: (i, 0)),
                  pl.BlockSpec((C, C), lambda i: (0, 0)),
                  pl.BlockSpec((C, C), lambda i: (0, 0))],
        out_specs=pl.BlockSpec((BN_ROWS, C), lambda i: (i, 0)),
        out_shape=jax.ShapeDtypeStruct((N, C), jnp.float32),
    )(x, w0, w1)


def _nt_body(h_ref, wd_ref, ws_ref, td_ref, ts_ref):
    h = h_ref[...]
    td_ref[...] = h @ wd_ref[...]
    ts_ref[...] = h @ ws_ref[...]


def _node_transform(h, wd, ws):
    return pl.pallas_call(
        _nt_body,
        grid=(NB_N,),
        in_specs=[pl.BlockSpec((BN_ROWS, C), lambda i: (i, 0)),
                  pl.BlockSpec((C, 2 * C), lambda i: (0, 0)),
                  pl.BlockSpec((C, 2 * C), lambda i: (0, 0))],
        out_specs=(pl.BlockSpec((BN_ROWS, 2 * C), lambda i: (i, 0)),
                   pl.BlockSpec((BN_ROWS, 2 * C), lambda i: (i, 0))),
        out_shape=(jax.ShapeDtypeStruct((N, 2 * C), jnp.float32),
                   jax.ShapeDtypeStruct((N, 2 * C), jnp.float32)),
    )(h, wd, ws)


def _ew_body(gd_ref, gs_ref, ea_ref, wfe_ref, wse_ref, bf_ref, bs_ref, o_ref):
    gd = gd_ref[...]
    gs = gs_ref[...]
    ea = ea_ref[...]
    f = gd[:, :C] + gs[:, :C] + ea @ wfe_ref[...] + bf_ref[...]
    s = gd[:, C:] + gs[:, C:] + ea @ wse_ref[...] + bs_ref[...]
    sig = 1.0 / (1.0 + jnp.exp(-f))
    sp = jnp.maximum(s, 0.0) + jnp.log(1.0 + jnp.exp(-jnp.abs(s)))
    o_ref[...] = sig * sp


def _edge_msg(gd, gs, ea, wfe, wse, bf, bs):
    return pl.pallas_call(
        _ew_body,
        grid=(NB_E,),
        in_specs=[pl.BlockSpec((BE_ROWS, 2 * C), lambda i: (i, 0)),
                  pl.BlockSpec((BE_ROWS, 2 * C), lambda i: (i, 0)),
                  pl.BlockSpec((BE_ROWS, DE), lambda i: (i, 0)),
                  pl.BlockSpec((DE, C), lambda i: (0, 0)),
                  pl.BlockSpec((DE, C), lambda i: (0, 0)),
                  pl.BlockSpec((1, C), lambda i: (0, 0)),
                  pl.BlockSpec((1, C), lambda i: (0, 0))],
        out_specs=pl.BlockSpec((BE_ROWS, C), lambda i: (i, 0)),
        out_shape=jax.ShapeDtypeStruct((E, C), jnp.float32),
    )(gd, gs, ea, wfe, wse, bf, bs)


def _agg_body(p0_ref, p1_ref, agg_ref, st_ref):
    a = p0_ref[...] + p1_ref[...]
    agg_ref[...] = a

    @pl.when(pl.program_id(0) == 0)
    def _():
        st_ref[...] = jnp.zeros_like(st_ref)

    s = jnp.sum(a, axis=0, keepdims=True)
    q = jnp.sum(a * a, axis=0, keepdims=True)
    st_ref[...] += jnp.concatenate([s, q, jnp.zeros((6, C), jnp.float32)], axis=0)


def _agg_stats(parts):
    return pl.pallas_call(
        _agg_body,
        grid=(NB_N,),
        in_specs=[pl.BlockSpec((BN_ROWS, C), lambda i: (i, 0)),
                  pl.BlockSpec((BN_ROWS, C), lambda i: (i + NB_N, 0))],
        out_specs=(pl.BlockSpec((BN_ROWS, C), lambda i: (i, 0)),
                   pl.BlockSpec((8, C), lambda i: (0, 0))),
        out_shape=(jax.ShapeDtypeStruct((N, C), jnp.float32),
                   jax.ShapeDtypeStruct((8, C), jnp.float32)),
    )(parts, parts)


def _bn_body(agg_ref, h_ref, st_ref, gam_ref, bet_ref, b_ref, hn_ref, gf_ref):
    st = st_ref[...]
    mean = st[0:1, :] * (1.0 / N)
    var = st[1:2, :] * (1.0 / N) - mean * mean
    a = agg_ref[...]
    o = (a - mean) / jnp.sqrt(var + 1e-5) * gam_ref[...] + bet_ref[...] + h_ref[...]
    hn = jnp.maximum(o, 0.0)
    hn_ref[...] = hn

    @pl.when(pl.program_id(0) == 0)
    def _():
        gf_ref[...] = jnp.full((G, C), -jnp.inf, jnp.float32)

    b = b_ref[...]
    ms = [jnp.max(jnp.where(b == g, hn, -jnp.inf), axis=0, keepdims=True)
          for g in range(G)]
    gf_ref[...] = jnp.maximum(gf_ref[...], jnp.concatenate(ms, axis=0))


def _bn_relu_segmax(agg, h, st, gamma, beta, batch_f):
    return pl.pallas_call(
        _bn_body,
        grid=(NB_N,),
        in_specs=[pl.BlockSpec((BN_ROWS, C), lambda i: (i, 0)),
                  pl.BlockSpec((BN_ROWS, C), lambda i: (i, 0)),
                  pl.BlockSpec((8, C), lambda i: (0, 0)),
                  pl.BlockSpec((1, C), lambda i: (0, 0)),
                  pl.BlockSpec((1, C), lambda i: (0, 0)),
                  pl.BlockSpec((BN_ROWS, 1), lambda i: (i, 0))],
        out_specs=(pl.BlockSpec((BN_ROWS, C), lambda i: (i, 0)),
                   pl.BlockSpec((G, C), lambda i: (0, 0))),
        out_shape=(jax.ShapeDtypeStruct((N, C), jnp.float32),
                   jax.ShapeDtypeStruct((G, C), jnp.float32)),
    )(agg, h, st, gamma, beta, batch_f)


def _head_body(g0_ref, g1_ref, g2_ref, w1_ref, b1_ref, gam_ref, bet_ref,
               w2_ref, b2_ref, o_ref):
    gf = g0_ref[...] + g1_ref[...] + g2_ref[...]
    g = gf @ w1_ref[...] + b1_ref[...]
    m = jnp.mean(g, axis=0, keepdims=True)
    v = jnp.mean((g - m) ** 2, axis=0, keepdims=True)
    gr = jnp.maximum((g - m) / jnp.sqrt(v + 1e-5) * gam_ref[...] + bet_ref[...], 0.0)
    r = jnp.sum(gr * w2_ref[...], axis=1, keepdims=True) + b2_ref[0:1, 0:1]
    o_ref[...] = jnp.broadcast_to(r, (G, C))


def _head(g0, g1, g2, w1, b1, gam, bet, w2row, b2):
    full = lambda shape: pl.BlockSpec(shape, lambda: (0, 0))
    return pl.pallas_call(
        _head_body,
        in_specs=[full((G, C)), full((G, C)), full((G, C)), full((C, C)),
                  full((1, C)), full((1, C)), full((1, C)), full((1, C)),
                  full((8, C))],
        out_specs=full((G, C)),
        out_shape=jax.ShapeDtypeStruct((G, C), jnp.float32),
    )(g0, g1, g2, w1, b1, gam, bet, w2row, b2)


_MESH = plsc.VectorSubcoreMesh(core_axis_name="c", subcore_axis_name="s")


@functools.partial(
    pl.kernel,
    mesh=_MESH,
    out_type=(jax.ShapeDtypeStruct((E, 2 * C), jnp.float32),
              jax.ShapeDtypeStruct((E, 2 * C), jnp.float32)),
    scratch_types=[
        pltpu.VMEM((K,), jnp.int32),
        pltpu.VMEM((K,), jnp.int32),
        pltpu.VMEM((K, 2 * C), jnp.float32),
        pltpu.VMEM((K, 2 * C), jnp.float32),
        pltpu.SemaphoreType.DMA,
        pltpu.SemaphoreType.DMA,
    ],
)
def _gather(td, ts, dst, src, gd, gs, idx_d, idx_s, rows_d, rows_s, sem_d, sem_s):
    cid = lax.axis_index("c")
    sid = lax.axis_index("s")
    wid = sid * 2 + cid

    def body(i, carry):
        base = wid * EPW + i * K
        pltpu.sync_copy(dst.at[pl.ds(base, K)], idx_d)
        pltpu.sync_copy(src.at[pl.ds(base, K)], idx_s)
        cp_d = pltpu.async_copy(td.at[idx_d], rows_d, sem_d)
        cp_s = pltpu.async_copy(ts.at[idx_s], rows_s, sem_s)
        cp_d.wait()
        cp_s.wait()
        pltpu.sync_copy(rows_d, gd.at[pl.ds(base, K)])
        pltpu.sync_copy(rows_s, gs.at[pl.ds(base, K)])
        return carry

    lax.fori_loop(0, NCH, body, 0)


@functools.partial(
    pl.kernel,
    mesh=_MESH,
    out_type=jax.ShapeDtypeStruct((2 * N, C), jnp.float32),
    scratch_types=[
        pltpu.VMEM((K,), jnp.int32),
        pltpu.VMEM((K, C), jnp.float32),
        pltpu.VMEM_SHARED((N, C), jnp.float32),
    ],
)
def _scatter(msg, dst, zeros, out, idx_v, rows_v, acc):
    cid = lax.axis_index("c")
    sid = lax.axis_index("s")
    pltpu.sync_copy(zeros.at[pl.ds(sid * RPT, RPT)], acc.at[pl.ds(sid * RPT, RPT)])

    @pl.when(sid == 0)
    def _():
        pltpu.sync_copy(zeros.at[pl.ds(16 * RPT, RTAIL)],
                        acc.at[pl.ds(16 * RPT, RTAIL)])

    plsc.subcore_barrier()

    def body(i, carry):
        base = cid * (E // 2) + sid * EPW + i * K
        pltpu.sync_copy(dst.at[pl.ds(base, K)], idx_v)
        pltpu.sync_copy(msg.at[pl.ds(base, K)], rows_v)
        pltpu.sync_copy(rows_v, acc.at[idx_v], add=True)
        return carry

    lax.fori_loop(0, NCH, body, 0)
    plsc.subcore_barrier()
    pltpu.sync_copy(acc.at[pl.ds(sid * RPT, RPT)],
                    out.at[pl.ds(cid * N + sid * RPT, RPT)])

    @pl.when(sid == 0)
    def _():
        pltpu.sync_copy(acc.at[pl.ds(16 * RPT, RTAIL)],
                        out.at[pl.ds(cid * N + 16 * RPT, RTAIL)])


def kernel(x, edge_index, edge_attr, batch, params):
    src = edge_index[0].astype(jnp.int32)
    dst = edge_index[1].astype(jnp.int32)
    batch_f = batch.astype(jnp.float32).reshape(N, 1)
    zeros = jnp.zeros((N, C), jnp.float32)

    h = _embed(x, params['emb_W0'], params['emb_W1'])
    gfs = []
    for l in range(L):
        wf = params[f'conv{l}_Wf']
        ws = params[f'conv{l}_Ws']
        wd = jnp.concatenate([wf[:C], ws[:C]], axis=1)
        wsrc = jnp.concatenate([wf[C:2 * C], ws[C:2 * C]], axis=1)
        td, ts = _node_transform(h, wd, wsrc)
        gd, gs = _gather(td, ts, dst, src)
        msg = _edge_msg(gd, gs, edge_attr, wf[2 * C:], ws[2 * C:],
                        params[f'conv{l}_bf'].reshape(1, C),
                        params[f'conv{l}_bs'].reshape(1, C))
        parts = _scatter(msg, dst, zeros)
        agg, st = _agg_stats(parts)
        h, gf = _bn_relu_segmax(agg, h, st,
                                params[f'conv{l}_gamma'].reshape(1, C),
                                params[f'conv{l}_beta'].reshape(1, C),
                                batch_f)
        gfs.append(gf)

    out = _head(gfs[0], gfs[1], gfs[2], params['lin1_W'],
                params['lin1_b'].reshape(1, C),
                params['bn_gamma'].reshape(1, C),
                params['bn_beta'].reshape(1, C),
                params['lin2_W'].reshape(1, C),
                jnp.broadcast_to(params['lin2_b'].reshape(1, 1), (8, C)))
    return out[:, 0]


# speedup vs baseline: 2.2167x; 1.2690x over previous
import functools

import jax
import jax.numpy as jnp
from jax import lax
from jax.experimental import pallas as pl
from jax.experimental.pallas import tpu as pltpu
from jax.experimental.pallas import tpu_sc as plsc

N = 10000
E = 320000
C = 128
DE = 16
G = 64
L = 3

NB_N = 10
BN_ROWS = N // NB_N
BE_ROWS = 1000
NB_E = E // BE_ROWS

NW = 32
EPW = E // NW
K = 80
NCH = EPW // K
RPT = 624
RTAIL = N - 16 * RPT


def _embed_body(x_ref, w0_ref, w1_ref, o_ref):
    h = jnp.maximum(x_ref[...] @ w0_ref[...], 0.0)
    o_ref[...] = h @ w1_ref[...]


def _embed(x, w0, w1):
    return pl.pallas_call(
        _embed_body,
        grid=(NB_N,),
        in_specs=[pl.BlockSpec((BN_ROWS, C), lambda i: (i, 0)),
                  pl.BlockSpec((C, C), lambda i: (0, 0)),
                  pl.BlockSpec((C, C), lambda i: (0, 0))],
        out_specs=pl.BlockSpec((BN_ROWS, C), lambda i: (i, 0)),
        out_shape=jax.ShapeDtypeStruct((N, C), jnp.float32),
    )(x, w0, w1)


def _nt_body(h_ref, wd_ref, ws_ref, td_ref, ts_ref):
    h = h_ref[...]
    td_ref[...] = h @ wd_ref[...]
    ts_ref[...] = h @ ws_ref[...]


def _node_transform(h, wd, ws):
    return pl.pallas_call(
        _nt_body,
        grid=(NB_N,),
        in_specs=[pl.BlockSpec((BN_ROWS, C), lambda i: (i, 0)),
                  pl.BlockSpec((C, 2 * C), lambda i: (0, 0)),
                  pl.BlockSpec((C, 2 * C), lambda i: (0, 0))],
        out_specs=(pl.BlockSpec((BN_ROWS, 2 * C), lambda i: (i, 0)),
                   pl.BlockSpec((BN_ROWS, 2 * C), lambda i: (i, 0))),
        out_shape=(jax.ShapeDtypeStruct((N, 2 * C), jnp.float32),
                   jax.ShapeDtypeStruct((N, 2 * C), jnp.float32)),
    )(h, wd, ws)


def _ew_body(gd_ref, gs_ref, ea_ref, wfe_ref, wse_ref, bf_ref, bs_ref, o_ref):
    gd = gd_ref[...]
    gs = gs_ref[...]
    ea = ea_ref[...]
    f = gd[:, :C] + gs[:, :C] + ea @ wfe_ref[...] + bf_ref[...]
    s = gd[:, C:] + gs[:, C:] + ea @ wse_ref[...] + bs_ref[...]
    sig = 1.0 / (1.0 + jnp.exp(-f))
    sp = jnp.maximum(s, 0.0) + jnp.log(1.0 + jnp.exp(-jnp.abs(s)))
    o_ref[...] = sig * sp


def _edge_msg(gd, gs, ea, wfe, wse, bf, bs):
    return pl.pallas_call(
        _ew_body,
        grid=(NB_E,),
        in_specs=[pl.BlockSpec((BE_ROWS, 2 * C), lambda i: (i, 0)),
                  pl.BlockSpec((BE_ROWS, 2 * C), lambda i: (i, 0)),
                  pl.BlockSpec((BE_ROWS, DE), lambda i: (i, 0)),
                  pl.BlockSpec((DE, C), lambda i: (0, 0)),
                  pl.BlockSpec((DE, C), lambda i: (0, 0)),
                  pl.BlockSpec((1, C), lambda i: (0, 0)),
                  pl.BlockSpec((1, C), lambda i: (0, 0))],
        out_specs=pl.BlockSpec((BE_ROWS, C), lambda i: (i, 0)),
        out_shape=jax.ShapeDtypeStruct((E, C), jnp.float32),
    )(gd, gs, ea, wfe, wse, bf, bs)


def _agg_body(p0_ref, p1_ref, agg_ref, st_ref):
    a = p0_ref[...] + p1_ref[...]
    agg_ref[...] = a

    @pl.when(pl.program_id(0) == 0)
    def _():
        st_ref[...] = jnp.zeros_like(st_ref)

    s = jnp.sum(a, axis=0, keepdims=True)
    q = jnp.sum(a * a, axis=0, keepdims=True)
    st_ref[...] += jnp.concatenate([s, q, jnp.zeros((6, C), jnp.float32)], axis=0)


def _agg_stats(parts):
    return pl.pallas_call(
        _agg_body,
        grid=(NB_N,),
        in_specs=[pl.BlockSpec((BN_ROWS, C), lambda i: (i, 0)),
                  pl.BlockSpec((BN_ROWS, C), lambda i: (i + NB_N, 0))],
        out_specs=(pl.BlockSpec((BN_ROWS, C), lambda i: (i, 0)),
                   pl.BlockSpec((8, C), lambda i: (0, 0))),
        out_shape=(jax.ShapeDtypeStruct((N, C), jnp.float32),
                   jax.ShapeDtypeStruct((8, C), jnp.float32)),
    )(parts, parts)


def _bn_body(agg_ref, h_ref, st_ref, gam_ref, bet_ref, b_ref, hn_ref, gf_ref):
    st = st_ref[...]
    mean = st[0:1, :] * (1.0 / N)
    var = st[1:2, :] * (1.0 / N) - mean * mean
    a = agg_ref[...]
    o = (a - mean) / jnp.sqrt(var + 1e-5) * gam_ref[...] + bet_ref[...] + h_ref[...]
    hn = jnp.maximum(o, 0.0)
    hn_ref[...] = hn

    @pl.when(pl.program_id(0) == 0)
    def _():
        gf_ref[...] = jnp.full((G, C), -jnp.inf, jnp.float32)

    b = b_ref[...]
    ms = [jnp.max(jnp.where(b == g, hn, -jnp.inf), axis=0, keepdims=True)
          for g in range(G)]
    gf_ref[...] = jnp.maximum(gf_ref[...], jnp.concatenate(ms, axis=0))


def _bn_relu_segmax(agg, h, st, gamma, beta, batch_f):
    return pl.pallas_call(
        _bn_body,
        grid=(NB_N,),
        in_specs=[pl.BlockSpec((BN_ROWS, C), lambda i: (i, 0)),
                  pl.BlockSpec((BN_ROWS, C), lambda i: (i, 0)),
                  pl.BlockSpec((8, C), lambda i: (0, 0)),
                  pl.BlockSpec((1, C), lambda i: (0, 0)),
                  pl.BlockSpec((1, C), lambda i: (0, 0)),
                  pl.BlockSpec((BN_ROWS, 1), lambda i: (i, 0))],
        out_specs=(pl.BlockSpec((BN_ROWS, C), lambda i: (i, 0)),
                   pl.BlockSpec((G, C), lambda i: (0, 0))),
        out_shape=(jax.ShapeDtypeStruct((N, C), jnp.float32),
                   jax.ShapeDtypeStruct((G, C), jnp.float32)),
    )(agg, h, st, gamma, beta, batch_f)


def _head_body(g0_ref, g1_ref, g2_ref, w1_ref, b1_ref, gam_ref, bet_ref,
               w2_ref, b2_ref, o_ref):
    gf = g0_ref[...] + g1_ref[...] + g2_ref[...]
    g = gf @ w1_ref[...] + b1_ref[...]
    m = jnp.mean(g, axis=0, keepdims=True)
    v = jnp.mean((g - m) ** 2, axis=0, keepdims=True)
    gr = jnp.maximum((g - m) / jnp.sqrt(v + 1e-5) * gam_ref[...] + bet_ref[...], 0.0)
    r = jnp.sum(gr * w2_ref[...], axis=1, keepdims=True) + b2_ref[0:1, 0:1]
    o_ref[...] = jnp.broadcast_to(r, (G, C))


def _head(g0, g1, g2, w1, b1, gam, bet, w2row, b2):
    full = lambda shape: pl.BlockSpec(shape, lambda: (0, 0))
    return pl.pallas_call(
        _head_body,
        in_specs=[full((G, C)), full((G, C)), full((G, C)), full((C, C)),
                  full((1, C)), full((1, C)), full((1, C)), full((1, C)),
                  full((8, C))],
        out_specs=full((G, C)),
        out_shape=jax.ShapeDtypeStruct((G, C), jnp.float32),
    )(g0, g1, g2, w1, b1, gam, bet, w2row, b2)


_MESH = plsc.VectorSubcoreMesh(core_axis_name="c", subcore_axis_name="s")


@functools.partial(
    pl.kernel,
    mesh=_MESH,
    out_type=(jax.ShapeDtypeStruct((E, 2 * C), jnp.float32),
              jax.ShapeDtypeStruct((E, 2 * C), jnp.float32)),
    scratch_types=[
        pltpu.VMEM((EPW,), jnp.int32),
        pltpu.VMEM((EPW,), jnp.int32),
        pltpu.VMEM((K, 2 * C), jnp.float32),
        pltpu.VMEM((K, 2 * C), jnp.float32),
        pltpu.VMEM((K, 2 * C), jnp.float32),
        pltpu.VMEM((K, 2 * C), jnp.float32),
        pltpu.SemaphoreType.DMA,
        pltpu.SemaphoreType.DMA,
    ],
)
def _gather(td, ts, dst, src, gd, gs, idx_d, idx_s, rd0, rs0, rd1, rs1, g0, g1):
    cid = lax.axis_index("c")
    sid = lax.axis_index("s")
    wid = sid * 2 + cid
    tbase = wid * EPW
    pltpu.sync_copy(dst.at[pl.ds(tbase, EPW)], idx_d)
    pltpu.sync_copy(src.at[pl.ds(tbase, EPW)], idx_s)

    def issue(c, rd, rs, sem):
        pltpu.async_copy(td.at[idx_d.at[pl.ds(c * K, K)]], rd, sem)
        pltpu.async_copy(ts.at[idx_s.at[pl.ds(c * K, K)]], rs, sem)

    def finish(c, rd, rs, sem):
        pltpu.make_async_copy(td.at[idx_d.at[pl.ds(0, K)]], rd, sem).wait()
        pltpu.make_async_copy(ts.at[idx_s.at[pl.ds(0, K)]], rs, sem).wait()
        pltpu.sync_copy(rd, gd.at[pl.ds(tbase + c * K, K)])
        pltpu.sync_copy(rs, gs.at[pl.ds(tbase + c * K, K)])

    issue(0, rd0, rs0, g0)
    issue(1, rd1, rs1, g1)

    def body(j, carry):
        c0 = 2 * j
        finish(c0, rd0, rs0, g0)

        @pl.when(c0 + 2 < NCH)
        def _():
            issue(c0 + 2, rd0, rs0, g0)

        @pl.when(c0 + 1 < NCH)
        def _():
            finish(c0 + 1, rd1, rs1, g1)

            @pl.when(c0 + 3 < NCH)
            def _():
                issue(c0 + 3, rd1, rs1, g1)

        return carry

    lax.fori_loop(0, (NCH + 1) // 2, body, 0)


@functools.partial(
    pl.kernel,
    mesh=_MESH,
    out_type=jax.ShapeDtypeStruct((2 * N, C), jnp.float32),
    scratch_types=[
        pltpu.VMEM((NCH, K), jnp.int32),
        pltpu.VMEM((K, C), jnp.float32),
        pltpu.VMEM((K, C), jnp.float32),
        pltpu.VMEM_SHARED((N, C), jnp.float32),
        pltpu.SemaphoreType.DMA,
        pltpu.SemaphoreType.DMA,
    ],
)
def _scatter(msg, dst_r, zeros, out, idx_v, m0, m1, acc, s0, s1):
    cid = lax.axis_index("c")
    sid = lax.axis_index("s")
    pltpu.sync_copy(dst_r.at[cid, sid], idx_v)
    pltpu.sync_copy(zeros.at[pl.ds(sid * RPT, RPT)], acc.at[pl.ds(sid * RPT, RPT)])

    @pl.when(sid == 0)
    def _():
        pltpu.sync_copy(zeros.at[pl.ds(16 * RPT, RTAIL)],
                        acc.at[pl.ds(16 * RPT, RTAIL)])

    plsc.subcore_barrier()
    ebase = cid * (E // 2) + sid * EPW

    def load(c, m, sem):
        pltpu.async_copy(msg.at[pl.ds(ebase + c * K, K)], m, sem)

    def flush(c, m, sem):
        pltpu.make_async_copy(msg.at[pl.ds(ebase, K)], m, sem).wait()
        pltpu.sync_copy(m, acc.at[idx_v.at[c]], add=True)

    load(0, m0, s0)
    load(1, m1, s1)

    def body(j, carry):
        c0 = 2 * j
        flush(c0, m0, s0)

        @pl.when(c0 + 2 < NCH)
        def _():
            load(c0 + 2, m0, s0)

        @pl.when(c0 + 1 < NCH)
        def _():
            flush(c0 + 1, m1, s1)

            @pl.when(c0 + 3 < NCH)
            def _():
                load(c0 + 3, m1, s1)

        return carry

    lax.fori_loop(0, (NCH + 1) // 2, body, 0)
    plsc.subcore_barrier()
    pltpu.sync_copy(acc.at[pl.ds(sid * RPT, RPT)],
                    out.at[pl.ds(cid * N + sid * RPT, RPT)])

    @pl.when(sid == 0)
    def _():
        pltpu.sync_copy(acc.at[pl.ds(16 * RPT, RTAIL)],
                        out.at[pl.ds(cid * N + 16 * RPT, RTAIL)])


def kernel(x, edge_index, edge_attr, batch, params):
    src = edge_index[0].astype(jnp.int32)
    dst = edge_index[1].astype(jnp.int32)
    dst_r = dst.reshape(2, 16, NCH, K)
    batch_f = batch.astype(jnp.float32).reshape(N, 1)
    zeros = jnp.zeros((N, C), jnp.float32)

    h = _embed(x, params['emb_W0'], params['emb_W1'])
    gfs = []
    for l in range(L):
        wf = params[f'conv{l}_Wf']
        ws = params[f'conv{l}_Ws']
        wd = jnp.concatenate([wf[:C], ws[:C]], axis=1)
        wsrc = jnp.concatenate([wf[C:2 * C], ws[C:2 * C]], axis=1)
        td, ts = _node_transform(h, wd, wsrc)
        gd, gs = _gather(td, ts, dst, src)
        msg = _edge_msg(gd, gs, edge_attr, wf[2 * C:], ws[2 * C:],
                        params[f'conv{l}_bf'].reshape(1, C),
                        params[f'conv{l}_bs'].reshape(1, C))
        parts = _scatter(msg, dst_r, zeros)
        agg, st = _agg_stats(parts)
        h, gf = _bn_relu_segmax(agg, h, st,
                                params[f'conv{l}_gamma'].reshape(1, C),
                                params[f'conv{l}_beta'].reshape(1, C),
                                batch_f)
        gfs.append(gf)

    out = _head(gfs[0], gfs[1], gfs[2], params['lin1_W'],
                params['lin1_b'].reshape(1, C),
                params['bn_gamma'].reshape(1, C),
                params['bn_beta'].reshape(1, C),
                params['lin2_W'].reshape(1, C),
                jnp.broadcast_to(params['lin2_b'].reshape(1, 1), (8, C)))
    return out[:, 0]
